# Initial kernel scaffold; baseline (speedup 1.0000x reference)
import jax
import jax.numpy as jnp
from jax.experimental import pallas as pl


def kernel(indices, embeddings):
    raise NotImplementedError("write your pallas kernel here")


# speedup vs baseline: 1.0223x; 1.0223x over previous
import functools

import jax
import jax.numpy as jnp
from jax import lax
from jax.experimental import pallas as pl
from jax.experimental.pallas import tpu as pltpu
from jax.experimental.pallas import tpu_sc as plsc

NUM_NODES = 1000000
EMBED_DIM = 32
BATCH = 16384
HIST = 50

_B_TOTAL = BATCH * HIST
_NC, _NS = 2, 16
_NW = _NC * _NS
_CHUNK = 128
_PER_W = _B_TOTAL // _NW
_NCHUNK = _PER_W // _CHUNK


def _gather_body(idx_hbm, table_hbm, out_hbm, idx_v, rows_v, sem):
    wid = lax.axis_index("s") * _NC + lax.axis_index("c")
    chunk_base = wid * _NCHUNK
    row_base = wid * _PER_W

    pltpu.sync_copy(idx_hbm.at[pl.ds(chunk_base, _NCHUNK)], idx_v)

    def step(j, _):
        pltpu.async_copy(table_hbm.at[idx_v.at[j]], rows_v, sem).wait()
        pltpu.sync_copy(rows_v, out_hbm.at[pl.ds(row_base + j * _CHUNK, _CHUNK)])
        return 0

    lax.fori_loop(0, _NCHUNK, step, 0)


_sc_gather = pl.kernel(
    _gather_body,
    mesh=plsc.VectorSubcoreMesh(core_axis_name="c", subcore_axis_name="s"),
    out_type=jax.ShapeDtypeStruct((_B_TOTAL, EMBED_DIM), jnp.float32),
    scratch_types=[
        pltpu.VMEM((_NCHUNK, _CHUNK), jnp.int32),
        pltpu.VMEM((_CHUNK, EMBED_DIM), jnp.float32),
        pltpu.SemaphoreType.DMA,
    ],
    compiler_params=pltpu.CompilerParams(use_tc_tiling_on_sc=False),
)


def kernel(indices, embeddings):
    idx2d = indices.reshape(_B_TOTAL // _CHUNK, _CHUNK)
    flat = _sc_gather(idx2d, embeddings)
    return flat.reshape(BATCH, HIST, EMBED_DIM)


# speedup vs baseline: 1.1097x; 1.0855x over previous
import functools

import jax
import jax.numpy as jnp
from jax import lax
from jax.experimental import pallas as pl
from jax.experimental.pallas import tpu as pltpu
from jax.experimental.pallas import tpu_sc as plsc

NUM_NODES = 1000000
EMBED_DIM = 32
BATCH = 16384
HIST = 50

_B_TOTAL = BATCH * HIST
_NC, _NS = 2, 16
_NW = _NC * _NS
_CHUNK = 128
_PER_W = _B_TOTAL // _NW
_NCHUNK = _PER_W // _CHUNK
_K = 10
_GROUP = _K * _CHUNK
_NG = _NCHUNK // _K


def _gather_body(idx_hbm, table_hbm, out_hbm, idx_v, buf0, buf1,
                 gsem0, gsem1, osem0, osem1):
    wid = lax.axis_index("s") * _NC + lax.axis_index("c")
    chunk_base = wid * _NCHUNK
    row_base = wid * _PER_W

    pltpu.sync_copy(idx_hbm.at[pl.ds(chunk_base, _NCHUNK)], idx_v)

    def fire(g, buf, gsem):
        for j in range(_K):
            pltpu.async_copy(
                table_hbm.at[idx_v.at[g * _K + j]],
                buf.at[pl.ds(j * _CHUNK, _CHUNK)],
                gsem)

    def drain_gathers(buf, gsem):
        pltpu.make_async_copy(out_hbm.at[pl.ds(0, _GROUP)], buf, gsem).wait()

    def start_out(g, buf, osem):
        pltpu.async_copy(buf, out_hbm.at[pl.ds(row_base + g * _GROUP, _GROUP)],
                         osem)

    def wait_out(buf, osem):
        pltpu.make_async_copy(out_hbm.at[pl.ds(0, _GROUP)], buf, osem).wait()

    fire(0, buf0, gsem0)

    def outer(t, _):
        g0 = 2 * t
        g1 = 2 * t + 1
        drain_gathers(buf0, gsem0)

        @pl.when(t > 0)
        def _():
            wait_out(buf1, osem1)
        fire(g1, buf1, gsem1)
        start_out(g0, buf0, osem0)

        drain_gathers(buf1, gsem1)

        @pl.when(t < _NG // 2 - 1)
        def _():
            wait_out(buf0, osem0)
            fire(g0 + 2, buf0, gsem0)
        start_out(g1, buf1, osem1)
        return 0

    lax.fori_loop(0, _NG // 2, outer, 0)
    wait_out(buf0, osem0)
    wait_out(buf1, osem1)


_sc_gather = pl.kernel(
    _gather_body,
    mesh=plsc.VectorSubcoreMesh(core_axis_name="c", subcore_axis_name="s"),
    out_type=jax.ShapeDtypeStruct((_B_TOTAL, EMBED_DIM), jnp.float32),
    scratch_types=[
        pltpu.VMEM((_NCHUNK, _CHUNK), jnp.int32),
        pltpu.VMEM((_GROUP, EMBED_DIM), jnp.float32),
        pltpu.VMEM((_GROUP, EMBED_DIM), jnp.float32),
        pltpu.SemaphoreType.DMA,
        pltpu.SemaphoreType.DMA,
        pltpu.SemaphoreType.DMA,
        pltpu.SemaphoreType.DMA,
    ],
    compiler_params=pltpu.CompilerParams(use_tc_tiling_on_sc=False),
)


def kernel(indices, embeddings):
    idx2d = indices.reshape(_B_TOTAL // _CHUNK, _CHUNK)
    flat = _sc_gather(idx2d, embeddings)
    return flat.reshape(BATCH, HIST, EMBED_DIM)
